# Initial kernel scaffold; baseline (speedup 1.0000x reference)
import jax
import jax.numpy as jnp
from jax.experimental import pallas as pl


def kernel(cost, disparity_samples):
    raise NotImplementedError("write your pallas kernel here")


# speedup vs baseline: 7.2622x; 7.2622x over previous
import jax
import jax.numpy as jnp
from jax import lax
from jax.experimental import pallas as pl
from jax.experimental.pallas import tpu as pltpu
from jax.experimental.pallas import tpu_sc as plsc

B, C, H, W = 8, 24, 160, 320
NPIX = H * W
NW = 32
P = 1280
WPB = NW // B
SPAN = NPIX // WPB
K = SPAN // P
L = 16
NVEC = P // L


def _sc_top2(cost_hbm, ds_hbm, pred_hbm, prob_hbm, cost_v, ds_v, pred_v, prob_v):
    wid = lax.axis_index("s") * 2 + lax.axis_index("c")
    b = wid // WPB
    start = (wid % WPB) * SPAN

    def chunk_body(k, carry):
        base = pl.multiple_of(start + k * P, 128)
        pltpu.sync_copy(cost_hbm.at[b, :, pl.ds(base, P)], cost_v)
        pltpu.sync_copy(ds_hbm.at[b, :, pl.ds(base, P)], ds_v)

        def vec_body(i, c2):
            sl = pl.ds(i * L, L)
            v0 = cost_v[0, sl]
            dv0 = ds_v[0, sl]
            v1 = cost_v[1, sl]
            dv1 = ds_v[1, sl]
            gt = v1 > v0
            m1 = jnp.where(gt, v1, v0)
            d1 = jnp.where(gt, dv1, dv0)
            m2 = jnp.where(gt, v0, v1)
            d2 = jnp.where(gt, dv0, dv1)
            for c in range(2, C):
                v = cost_v[c, sl]
                dv = ds_v[c, sl]
                gt1 = v > m1
                gt2 = v > m2
                nm2 = jnp.where(gt1, m1, jnp.where(gt2, v, m2))
                nd2 = jnp.where(gt1, d1, jnp.where(gt2, dv, d2))
                m1 = jnp.where(gt1, v, m1)
                d1 = jnp.where(gt1, dv, d1)
                m2 = nm2
                d2 = nd2
            e = jnp.exp(m2 - m1)
            den = 1.0 + e
            p1 = e / den
            p0 = 1.0 / den
            pred_v[sl] = d1 * p0 + d2 * p1
            prob_v[0, sl] = p0
            prob_v[1, sl] = p1
            return c2

        lax.fori_loop(0, NVEC, vec_body, 0)
        pltpu.sync_copy(pred_v, pred_hbm.at[b, 0, pl.ds(base, P)])
        pltpu.sync_copy(prob_v, prob_hbm.at[b, :, pl.ds(base, P)])
        return carry

    lax.fori_loop(0, K, chunk_body, 0)


def kernel(cost, disparity_samples):
    cost3 = cost.reshape(B, C, NPIX)
    ds3 = disparity_samples.reshape(B, C, NPIX)
    mesh = plsc.VectorSubcoreMesh(core_axis_name="c", subcore_axis_name="s")
    f = pl.kernel(
        _sc_top2,
        mesh=mesh,
        out_type=[
            jax.ShapeDtypeStruct((B, 1, NPIX), jnp.float32),
            jax.ShapeDtypeStruct((B, 2, NPIX), jnp.float32),
        ],
        scratch_types=[
            pltpu.VMEM((C, P), jnp.float32),
            pltpu.VMEM((C, P), jnp.float32),
            pltpu.VMEM((P,), jnp.float32),
            pltpu.VMEM((2, P), jnp.float32),
        ],
    )
    pred, prob = f(cost3, ds3)
    return pred.reshape(B, H, W), prob.reshape(B, 2, H, W)


# speedup vs baseline: 7.6518x; 1.0536x over previous
import jax
import jax.numpy as jnp
from jax import lax
from jax.experimental import pallas as pl
from jax.experimental.pallas import tpu as pltpu
from jax.experimental.pallas import tpu_sc as plsc

B, C, H, W = 8, 24, 160, 320
NPIX = H * W
NW = 32
P = 640
WPB = NW // B
SPAN = NPIX // WPB
K = SPAN // P
NPAIR = K // 2
L = 16
NVEC = P // L


def _sc_top2(cost_hbm, ds_hbm, pred_hbm, prob_hbm,
             cost_v0, ds_v0, cost_v1, ds_v1,
             pred_v0, prob_v0, pred_v1, prob_v1,
             sin0, sin1, sout0, sout1):
    wid = lax.axis_index("s") * 2 + lax.axis_index("c")
    b = wid // WPB
    start = (wid % WPB) * SPAN
    inbuf = [(cost_v0, ds_v0, sin0), (cost_v1, ds_v1, sin1)]
    outbuf = [(pred_v0, prob_v0, sout0), (pred_v1, prob_v1, sout1)]

    def base_of(k):
        return pl.multiple_of(start + k * P, 128)

    def in_descs(k, p):
        cv, dv, sem = inbuf[p]
        sl = pl.ds(base_of(k), P)
        return (pltpu.make_async_copy(cost_hbm.at[b, :, sl], cv, sem),
                pltpu.make_async_copy(ds_hbm.at[b, :, sl], dv, sem))

    def out_descs(k, p):
        pv, pbv, sem = outbuf[p]
        sl = pl.ds(base_of(k), P)
        return (pltpu.make_async_copy(pv, pred_hbm.at[b, 0, sl], sem),
                pltpu.make_async_copy(pbv, prob_hbm.at[b, :, sl], sem))

    def compute(p):
        cv, dv, _ = inbuf[p]
        pv, pbv, _ = outbuf[p]

        @plsc.parallel_loop(0, NVEC)
        def vec_body(i):
            sl = pl.ds(i * L, L)
            v0 = cv[0, sl]
            dv0 = dv[0, sl]
            v1 = cv[1, sl]
            dv1 = dv[1, sl]
            gt = v1 > v0
            m1 = jnp.where(gt, v1, v0)
            d1 = jnp.where(gt, dv1, dv0)
            m2 = jnp.where(gt, v0, v1)
            d2 = jnp.where(gt, dv0, dv1)
            for c in range(2, C):
                v = cv[c, sl]
                dvv = dv[c, sl]
                gt1 = v > m1
                gt2 = v > m2
                nm2 = jnp.where(gt1, m1, jnp.where(gt2, v, m2))
                nd2 = jnp.where(gt1, d1, jnp.where(gt2, dvv, d2))
                m1 = jnp.where(gt1, v, m1)
                d1 = jnp.where(gt1, dvv, d1)
                m2 = nm2
                d2 = nd2
            e = jnp.exp(m2 - m1)
            den = 1.0 + e
            p1 = e / den
            p0 = 1.0 / den
            pv[sl] = d1 * p0 + d2 * p1
            pbv[0, sl] = p0
            pbv[1, sl] = p1

    def half_round(k, p, first, last):
        for d in in_descs(k, p):
            d.wait()
        if not first:
            for d in out_descs(k - 2, p):
                d.wait()
        compute(p)
        for d in out_descs(k, p):
            d.start()
        if not last:
            for d in in_descs(k + 2, p):
                d.start()

    for d in in_descs(0, 0) + in_descs(1, 1):
        d.start()

    half_round(0, 0, True, False)
    half_round(1, 1, True, False)

    def pair_body(jp, carry):
        k0 = jp * 2
        half_round(k0, 0, False, False)
        half_round(k0 + 1, 1, False, False)
        return carry

    lax.fori_loop(1, NPAIR - 1, pair_body, 0)

    half_round(K - 2, 0, False, True)
    half_round(K - 1, 1, False, True)

    for d in out_descs(K - 2, 0) + out_descs(K - 1, 1):
        d.wait()


def kernel(cost, disparity_samples):
    cost3 = cost.reshape(B, C, NPIX)
    ds3 = disparity_samples.reshape(B, C, NPIX)
    mesh = plsc.VectorSubcoreMesh(core_axis_name="c", subcore_axis_name="s")
    f = pl.kernel(
        _sc_top2,
        mesh=mesh,
        out_type=[
            jax.ShapeDtypeStruct((B, 1, NPIX), jnp.float32),
            jax.ShapeDtypeStruct((B, 2, NPIX), jnp.float32),
        ],
        scratch_types=[
            pltpu.VMEM((C, P), jnp.float32),
            pltpu.VMEM((C, P), jnp.float32),
            pltpu.VMEM((C, P), jnp.float32),
            pltpu.VMEM((C, P), jnp.float32),
            pltpu.VMEM((P,), jnp.float32),
            pltpu.VMEM((2, P), jnp.float32),
            pltpu.VMEM((P,), jnp.float32),
            pltpu.VMEM((2, P), jnp.float32),
            pltpu.SemaphoreType.DMA,
            pltpu.SemaphoreType.DMA,
            pltpu.SemaphoreType.DMA,
            pltpu.SemaphoreType.DMA,
        ],
    )
    pred, prob = f(cost3, ds3)
    return pred.reshape(B, H, W), prob.reshape(B, 2, H, W)
